# Initial kernel scaffold; baseline (speedup 1.0000x reference)
import jax
import jax.numpy as jnp
from jax.experimental import pallas as pl


def kernel(x, edge_index, path_indices, W1, b1, W2, b2, Wm1, bm1, Wm2, bm2):
    raise NotImplementedError("write your pallas kernel here")


# speedup vs baseline: 12.3752x; 12.3752x over previous
import functools

import jax
import jax.numpy as jnp
from jax import lax
from jax.experimental import pallas as pl
from jax.experimental.pallas import tpu as pltpu
from jax.experimental.pallas import tpu_sc as plsc

_F32 = jnp.float32

_NC = 2
_NS = 16
_NW = _NC * _NS


def _mesh():
    return plsc.VectorSubcoreMesh(core_axis_name="c", subcore_axis_name="s")


@functools.lru_cache(maxsize=None)
def _deg_kernel(E, NP, K):
    EW = E // _NW
    RPT = NP // _NS

    @functools.partial(
        pl.kernel,
        mesh=_mesh(),
        out_type=jax.ShapeDtypeStruct((_NC, NP), _F32),
        scratch_types=[
            pltpu.VMEM((K,), jnp.int32),
            pltpu.VMEM((K,), _F32),
            pltpu.VMEM((RPT,), _F32),
            pltpu.VMEM_SHARED((NP,), _F32),
        ],
    )
    def deg(dst_hbm, out_hbm, idx_v, ones_v, zbuf, acc):
        c = lax.axis_index("c")
        s = lax.axis_index("s")
        wid = c * _NS + s

        def fill_ones(i, carry):
            ones_v[pl.ds(i * 16, 16)] = jnp.ones((16,), _F32)
            return carry

        lax.fori_loop(0, K // 16, fill_ones, 0)

        def fill_zero(i, carry):
            zbuf[pl.ds(i * 16, 16)] = jnp.zeros((16,), _F32)
            return carry

        lax.fori_loop(0, RPT // 16, fill_zero, 0)
        pltpu.sync_copy(zbuf, acc.at[pl.ds(s * RPT, RPT)])
        plsc.subcore_barrier()

        base = wid * EW

        def body(i, carry):
            pltpu.sync_copy(dst_hbm.at[pl.ds(base + i * K, K)], idx_v)
            pltpu.sync_copy(ones_v, acc.at[idx_v], add=True)
            return carry

        lax.fori_loop(0, EW // K, body, 0)
        plsc.subcore_barrier()
        pltpu.sync_copy(acc.at[pl.ds(s * RPT, RPT)],
                        out_hbm.at[c, pl.ds(s * RPT, RPT)])

    return deg


@functools.lru_cache(maxsize=None)
def _edge_kernel(E, NP, H, K):
    EW = E // _NW
    RPT = NP // _NS
    ZR = 128

    @functools.partial(
        pl.kernel,
        mesh=_mesh(),
        out_type=jax.ShapeDtypeStruct((_NC, NP, H), _F32),
        scratch_types=[
            pltpu.VMEM((K,), jnp.int32),
            pltpu.VMEM((K,), jnp.int32),
            pltpu.VMEM((K, H), _F32),
            pltpu.VMEM((ZR, H), _F32),
            pltpu.VMEM_SHARED((NP, H), _F32),
            pltpu.SemaphoreType.DMA,
        ],
    )
    def edge(src_hbm, dst_hbm, xs_hbm, out_hbm,
             src_v, dst_v, rows_v, zbuf, acc, sem):
        c = lax.axis_index("c")
        s = lax.axis_index("s")
        wid = c * _NS + s

        def fill_zero(i, carry):
            r = i // (H // 16)
            j = i % (H // 16)
            zbuf[r, pl.ds(j * 16, 16)] = jnp.zeros((16,), _F32)
            return carry

        lax.fori_loop(0, ZR * (H // 16), fill_zero, 0)

        def flush_zero(z, carry):
            pltpu.sync_copy(zbuf, acc.at[pl.ds(s * RPT + z * ZR, ZR)])
            return carry

        lax.fori_loop(0, RPT // ZR, flush_zero, 0)
        plsc.subcore_barrier()

        base = wid * EW

        def body(i, carry):
            off = base + i * K
            pltpu.sync_copy(src_hbm.at[pl.ds(off, K)], src_v)
            pltpu.sync_copy(dst_hbm.at[pl.ds(off, K)], dst_v)
            pltpu.async_copy(xs_hbm.at[src_v], rows_v, sem).wait()
            pltpu.sync_copy(rows_v, acc.at[dst_v], add=True)
            return carry

        lax.fori_loop(0, EW // K, body, 0)
        plsc.subcore_barrier()
        pltpu.sync_copy(acc.at[pl.ds(s * RPT, RPT)],
                        out_hbm.at[c, pl.ds(s * RPT, RPT)])

    return edge


@functools.lru_cache(maxsize=None)
def _pool_kernel(P, L, H, NP):
    PP = P // _NW

    @functools.partial(
        pl.kernel,
        mesh=_mesh(),
        out_type=jax.ShapeDtypeStruct((P, H), _F32),
        scratch_types=[
            pltpu.VMEM((PP * L,), jnp.int32),
            pltpu.VMEM((PP * L, H), _F32),
            pltpu.VMEM((PP, H), _F32),
            pltpu.SemaphoreType.DMA,
        ],
    )
    def pool(idx_hbm, h_hbm, out_hbm, idx_v, rows_v, pe_v, sem):
        c = lax.axis_index("c")
        s = lax.axis_index("s")
        wid = c * _NS + s
        pltpu.sync_copy(idx_hbm.at[pl.ds(wid * PP * L, PP * L)], idx_v)
        pltpu.async_copy(h_hbm.at[idx_v], rows_v, sem).wait()
        inv_l = jnp.float32(1.0 / L)
        for p in range(PP):
            for j in range(H // 16):
                acc = jnp.zeros((16,), _F32)
                for l in range(L):
                    acc = acc + rows_v[p * L + l, pl.ds(j * 16, 16)]
                pe_v[p, pl.ds(j * 16, 16)] = acc * inv_l
        pltpu.sync_copy(pe_v, out_hbm.at[pl.ds(wid * PP, PP)])

    return pool


def _tc_layer1(x_pad, W1, p0, p1, B):
    NP, F = x_pad.shape
    H = W1.shape[1]

    def body(x_ref, w_ref, p0_ref, p1_ref, dinv_ref, xs_ref):
        xw = jnp.dot(x_ref[...], w_ref[...], preferred_element_type=_F32)
        dv = lax.rsqrt(p0_ref[...] + p1_ref[...] + 1.0)
        dinv_ref[...] = dv
        xs_ref[...] = xw * dv

    return pl.pallas_call(
        body,
        grid=(NP // B,),
        in_specs=[
            pl.BlockSpec((B, F), lambda g: (g, 0)),
            pl.BlockSpec((F, H), lambda g: (0, 0)),
            pl.BlockSpec((B, 1), lambda g: (g, 0)),
            pl.BlockSpec((B, 1), lambda g: (g, 0)),
        ],
        out_specs=[
            pl.BlockSpec((B, 1), lambda g: (g, 0)),
            pl.BlockSpec((B, H), lambda g: (g, 0)),
        ],
        out_shape=[
            jax.ShapeDtypeStruct((NP, 1), _F32),
            jax.ShapeDtypeStruct((NP, H), _F32),
        ],
    )(x_pad, W1, p0, p1)


def _tc_layer2(q0, q1, xs1, dinv, b1, W2, B):
    NP, H = xs1.shape

    def body(q0_ref, q1_ref, xs_ref, dv_ref, b_ref, w_ref, out_ref):
        dv = dv_ref[...]
        h1 = jnp.maximum(dv * (q0_ref[...] + q1_ref[...] + xs_ref[...])
                         + b_ref[...], 0.0)
        out_ref[...] = jnp.dot(h1, w_ref[...],
                               preferred_element_type=_F32) * dv

    return pl.pallas_call(
        body,
        grid=(NP // B,),
        in_specs=[
            pl.BlockSpec((B, H), lambda g: (g, 0)),
            pl.BlockSpec((B, H), lambda g: (g, 0)),
            pl.BlockSpec((B, H), lambda g: (g, 0)),
            pl.BlockSpec((B, 1), lambda g: (g, 0)),
            pl.BlockSpec((1, H), lambda g: (0, 0)),
            pl.BlockSpec((H, H), lambda g: (0, 0)),
        ],
        out_specs=pl.BlockSpec((B, H), lambda g: (g, 0)),
        out_shape=jax.ShapeDtypeStruct((NP, H), _F32),
    )(q0, q1, xs1, dinv, b1, W2)


def _tc_final_h(r0, r1, xs2, dinv, b2, B):
    NP, H = xs2.shape

    def body(r0_ref, r1_ref, xs_ref, dv_ref, b_ref, out_ref):
        out_ref[...] = jnp.maximum(
            dv_ref[...] * (r0_ref[...] + r1_ref[...] + xs_ref[...])
            + b_ref[...], 0.0)

    return pl.pallas_call(
        body,
        grid=(NP // B,),
        in_specs=[
            pl.BlockSpec((B, H), lambda g: (g, 0)),
            pl.BlockSpec((B, H), lambda g: (g, 0)),
            pl.BlockSpec((B, H), lambda g: (g, 0)),
            pl.BlockSpec((B, 1), lambda g: (g, 0)),
            pl.BlockSpec((1, H), lambda g: (0, 0)),
        ],
        out_specs=pl.BlockSpec((B, H), lambda g: (g, 0)),
        out_shape=jax.ShapeDtypeStruct((NP, H), _F32),
    )(r0, r1, xs2, dinv, b2)


def _tc_head(pe, Wm1, bm1, wm2_row, bm2):
    P, H = pe.shape
    M = Wm1.shape[1]

    def body(pe_ref, w1_ref, b1_ref, w2_ref, b2_ref, out_ref):
        hid = jnp.maximum(
            jnp.dot(pe_ref[...], w1_ref[...], preferred_element_type=_F32)
            + b1_ref[...], 0.0)
        sc = jnp.sum(hid * w2_ref[...], axis=1, keepdims=True) + b2_ref[0, 0]
        m = jnp.max(sc)
        e = jnp.exp(sc - m)
        out_ref[...] = e / jnp.sum(e)

    return pl.pallas_call(
        body,
        out_shape=jax.ShapeDtypeStruct((P, 1), _F32),
    )(pe, Wm1, bm1, wm2_row, bm2)


def kernel(x, edge_index, path_indices, W1, b1, W2, b2, Wm1, bm1, Wm2, bm2):
    N, F = x.shape
    H = W1.shape[1]
    E = edge_index.shape[1]
    P, L = path_indices.shape
    M = Wm1.shape[1]

    K = 80
    B = 256
    step = _NS * 128
    NP = -(-N // step) * step
    assert E % (_NW * K) == 0 and P % _NW == 0

    src = edge_index[0]
    dst = edge_index[1]
    x_pad = jnp.pad(x, ((0, NP - N), (0, 0)))

    degp = _deg_kernel(E, NP, K)(dst)
    p0 = degp[0].reshape(NP, 1)
    p1 = degp[1].reshape(NP, 1)

    dinv, xs1 = _tc_layer1(x_pad, W1, p0, p1, B)

    edge = _edge_kernel(E, NP, H, K)
    q = edge(src, dst, xs1)
    xs2 = _tc_layer2(q[0], q[1], xs1, dinv, b1.reshape(1, H), W2, B)

    r = edge(src, dst, xs2)
    h2 = _tc_final_h(r[0], r[1], xs2, dinv, b2.reshape(1, H), B)

    pe = _pool_kernel(P, L, H, NP)(path_indices.reshape(-1), h2)

    out = _tc_head(pe, Wm1, bm1.reshape(1, M), Wm2.reshape(1, M),
                   bm2.reshape(1, 1))
    return out.reshape(P)
